# baseline (speedup 1.0000x reference)
import functools

import jax
import jax.numpy as jnp
from jax import lax
from jax.experimental import pallas as pl
from jax.experimental.pallas import tpu as pltpu
from jax.experimental.pallas import tpu_sc as plsc
from jax.experimental import layout as jex_layout

NC = 2
NS = 16
L = 16
NW = NC * NS

B = 4096
D = 32
SLAB = 128
RPS = SLAB // D
BPW = B // NW


def _complex_score_body(pos_hbm, er_hbm, ei_hbm, rr_hbm, ri_hbm, out_hbm,
                        pos_v, hs_v, rs_v, ts_v, ho_v, ro_v, to_v,
                        hre_v, him_v, rre_v, rim_v, tre_v, tim_v,
                        half_v, out_v, sems):
    wid = lax.axis_index("s") * NC + lax.axis_index("c")
    base = wid * BPW

    pltpu.sync_copy(pos_hbm.at[pl.ds(base * 3, BPW * 3)], pos_v)
    for g in range(BPW // L):
        rows3 = (g * L + lax.iota(jnp.int32, L)) * 3
        sl = pl.ds(g * L, L)
        for c, sdst, odst in ((0, hs_v, ho_v), (1, rs_v, ro_v),
                              (2, ts_v, to_v)):
            r = plsc.load_gather(pos_v, [rows3 + c])
            sdst[sl] = r >> 2
            odst[sl] = (r & 3) * D

    copies = [
        pltpu.async_copy(er_hbm.at[hs_v], hre_v, sems[0]),
        pltpu.async_copy(ei_hbm.at[hs_v], him_v, sems[1]),
        pltpu.async_copy(rr_hbm.at[rs_v], rre_v, sems[2]),
        pltpu.async_copy(ri_hbm.at[rs_v], rim_v, sems[3]),
        pltpu.async_copy(er_hbm.at[ts_v], tre_v, sems[4]),
        pltpu.async_copy(ei_hbm.at[ts_v], tim_v, sems[5]),
    ]
    for cp in copies:
        cp.wait()

    lane = lax.iota(jnp.int32, L)
    zero = jnp.zeros((L,), jnp.int32)

    def row_fn(i, carry):
        j = lax.rem(i, L)
        s16 = i - j
        mask = lane == j
        co_h = jnp.sum(jnp.where(mask, ho_v[pl.ds(s16, L)], zero))
        co_r = jnp.sum(jnp.where(mask, ro_v[pl.ds(s16, L)], zero))
        co_t = jnp.sum(jnp.where(mask, to_v[pl.ds(s16, L)], zero))
        hr0 = hre_v[i, pl.ds(co_h, L)]
        hr1 = hre_v[i, pl.ds(co_h + L, L)]
        hi0 = him_v[i, pl.ds(co_h, L)]
        hi1 = him_v[i, pl.ds(co_h + L, L)]
        rr0 = rre_v[i, pl.ds(co_r, L)]
        rr1 = rre_v[i, pl.ds(co_r + L, L)]
        ri0 = rim_v[i, pl.ds(co_r, L)]
        ri1 = rim_v[i, pl.ds(co_r + L, L)]
        tr0 = tre_v[i, pl.ds(co_t, L)]
        tr1 = tre_v[i, pl.ds(co_t + L, L)]
        ti0 = tim_v[i, pl.ds(co_t, L)]
        ti1 = tim_v[i, pl.ds(co_t + L, L)]
        s0 = (hr0 * rr0 - hi0 * ri0) * tr0 + (hr0 * ri0 + hi0 * rr0) * ti0
        s1 = (hr1 * rr1 - hi1 * ri1) * tr1 + (hr1 * ri1 + hi1 * rr1) * ti1
        half_v[pl.ds(i * L, L)] = s0 + s1
        return carry

    lax.fori_loop(0, BPW, row_fn, 0)

    for g in range(BPW // L):
        rows16 = (g * L + lax.iota(jnp.int32, L)) * L
        acc = plsc.load_gather(half_v, [rows16])
        for j in range(1, L):
            acc = acc + plsc.load_gather(half_v, [rows16 + j])
        out_v[pl.ds(g * L, L)] = acc

    pltpu.sync_copy(out_v, out_hbm.at[pl.ds(base, BPW)])


@jax.jit
def _complex_score(pos_sample, ent_embd, ent_embd_im, rel_embd, rel_embd_im):
    mesh = plsc.VectorSubcoreMesh(
        core_axis_name="c", subcore_axis_name="s",
        num_cores=NC, num_subcores=NS)
    run = pl.kernel(
        _complex_score_body,
        out_type=jax.ShapeDtypeStruct((B,), jnp.float32),
        mesh=mesh,
        scratch_types=[
            pltpu.VMEM((BPW * 3,), jnp.int32),
            pltpu.VMEM((BPW,), jnp.int32),
            pltpu.VMEM((BPW,), jnp.int32),
            pltpu.VMEM((BPW,), jnp.int32),
            pltpu.VMEM((BPW,), jnp.int32),
            pltpu.VMEM((BPW,), jnp.int32),
            pltpu.VMEM((BPW,), jnp.int32),
            pltpu.VMEM((BPW, SLAB), jnp.float32),
            pltpu.VMEM((BPW, SLAB), jnp.float32),
            pltpu.VMEM((BPW, SLAB), jnp.float32),
            pltpu.VMEM((BPW, SLAB), jnp.float32),
            pltpu.VMEM((BPW, SLAB), jnp.float32),
            pltpu.VMEM((BPW, SLAB), jnp.float32),
            pltpu.VMEM((BPW * L,), jnp.float32),
            pltpu.VMEM((BPW,), jnp.float32),
            [pltpu.SemaphoreType.DMA] * 6,
        ],
        compiler_params=pltpu.CompilerParams(needs_layout_passes=False),
    )

    def prep(t):
        t4 = t.reshape(B * 0 + 250000, SLAB)
        return jex_layout.with_layout_constraint(
            t4, jex_layout.Layout((1, 0), ((8, 128),)))

    return run(pos_sample.reshape(-1), prep(ent_embd), prep(ent_embd_im),
               prep(rel_embd), prep(rel_embd_im))


def kernel(pos_sample, ent_embd, ent_embd_im, rel_embd, rel_embd_im):
    score = _complex_score(pos_sample, ent_embd, ent_embd_im,
                           rel_embd, rel_embd_im)
    return score.reshape(B, 1)


# speedup vs baseline: 1.0054x; 1.0054x over previous
import functools

import jax
import jax.numpy as jnp
from jax import lax
from jax.experimental import pallas as pl
from jax.experimental.pallas import tpu as pltpu
from jax.experimental.pallas import tpu_sc as plsc

NC = 2
NS = 16
L = 16
NW = NC * NS

B = 4096
D = 32
BPW = B // NW


def _complex_score_body(pos_hbm, er_hbm, ei_hbm, rr_hbm, ri_hbm, out_hbm,
                        pos_v, hi_v, ri_v, ti_v,
                        hre_v, him_v, rre_v, rim_v, tre_v, tim_v,
                        half_v, out_v, sems):
    wid = lax.axis_index("s") * NC + lax.axis_index("c")
    base = wid * BPW

    pltpu.sync_copy(pos_hbm.at[pl.ds(base * 3, BPW * 3)], pos_v)
    for g in range(BPW // L):
        rows3 = (g * L + lax.iota(jnp.int32, L)) * 3
        for c, dst in ((0, hi_v), (1, ri_v), (2, ti_v)):
            dst[pl.ds(g * L, L)] = plsc.load_gather(pos_v, [rows3 + c])

    copies = [
        pltpu.async_copy(er_hbm.at[hi_v], hre_v, sems[0]),
        pltpu.async_copy(ei_hbm.at[hi_v], him_v, sems[1]),
        pltpu.async_copy(rr_hbm.at[ri_v], rre_v, sems[2]),
        pltpu.async_copy(ri_hbm.at[ri_v], rim_v, sems[3]),
        pltpu.async_copy(er_hbm.at[ti_v], tre_v, sems[4]),
        pltpu.async_copy(ei_hbm.at[ti_v], tim_v, sems[5]),
    ]
    for cp in copies:
        cp.wait()

    def row_fn(i, carry):
        hr0 = hre_v[i, pl.ds(0, L)]
        hr1 = hre_v[i, pl.ds(L, L)]
        hi0 = him_v[i, pl.ds(0, L)]
        hi1 = him_v[i, pl.ds(L, L)]
        rr0 = rre_v[i, pl.ds(0, L)]
        rr1 = rre_v[i, pl.ds(L, L)]
        ri0 = rim_v[i, pl.ds(0, L)]
        ri1 = rim_v[i, pl.ds(L, L)]
        tr0 = tre_v[i, pl.ds(0, L)]
        tr1 = tre_v[i, pl.ds(L, L)]
        ti0 = tim_v[i, pl.ds(0, L)]
        ti1 = tim_v[i, pl.ds(L, L)]
        s0 = (hr0 * rr0 - hi0 * ri0) * tr0 + (hr0 * ri0 + hi0 * rr0) * ti0
        s1 = (hr1 * rr1 - hi1 * ri1) * tr1 + (hr1 * ri1 + hi1 * rr1) * ti1
        half_v[pl.ds(i * L, L)] = s0 + s1
        return carry

    lax.fori_loop(0, BPW, row_fn, 0)

    for g in range(BPW // L):
        rows16 = (g * L + lax.iota(jnp.int32, L)) * L
        acc = plsc.load_gather(half_v, [rows16])
        for j in range(1, L):
            acc = acc + plsc.load_gather(half_v, [rows16 + j])
        out_v[pl.ds(g * L, L)] = acc

    pltpu.sync_copy(out_v, out_hbm.at[pl.ds(base, BPW)])


@jax.jit
def _complex_score(pos_sample, ent_embd, ent_embd_im, rel_embd, rel_embd_im):
    mesh = plsc.VectorSubcoreMesh(
        core_axis_name="c", subcore_axis_name="s",
        num_cores=NC, num_subcores=NS)
    run = pl.kernel(
        _complex_score_body,
        out_type=jax.ShapeDtypeStruct((B,), jnp.float32),
        mesh=mesh,
        scratch_types=[
            pltpu.VMEM((BPW * 3,), jnp.int32),
            pltpu.VMEM((BPW,), jnp.int32),
            pltpu.VMEM((BPW,), jnp.int32),
            pltpu.VMEM((BPW,), jnp.int32),
            pltpu.VMEM((BPW, D), jnp.float32),
            pltpu.VMEM((BPW, D), jnp.float32),
            pltpu.VMEM((BPW, D), jnp.float32),
            pltpu.VMEM((BPW, D), jnp.float32),
            pltpu.VMEM((BPW, D), jnp.float32),
            pltpu.VMEM((BPW, D), jnp.float32),
            pltpu.VMEM((BPW * L,), jnp.float32),
            pltpu.VMEM((BPW,), jnp.float32),
            [pltpu.SemaphoreType.DMA] * 6,
        ],
        compiler_params=pltpu.CompilerParams(
            needs_layout_passes=False, use_tc_tiling_on_sc=False),
    )
    return run(pos_sample.reshape(-1), ent_embd, ent_embd_im,
               rel_embd, rel_embd_im)


def kernel(pos_sample, ent_embd, ent_embd_im, rel_embd, rel_embd_im):
    score = _complex_score(pos_sample, ent_embd, ent_embd_im,
                           rel_embd, rel_embd_im)
    return score.reshape(B, 1)
